# speedup vs baseline: 1.2703x; 1.2703x over previous
import functools

import jax
import jax.numpy as jnp
from jax import lax
from jax.experimental import pallas as pl
from jax.experimental.pallas import tpu as pltpu
from jax.experimental.pallas import tpu_sc as plsc

B = 4
T = 2048
D = 128
ROWS = B * T
NC = 2
NS = 16
NW = NC * NS
RPW = ROWS // NW
GCHUNK = 128
NG = RPW // GCHUNK
LANES = 16


def _emb_body(idx_hbm, tok_hbm, pos_hbm, out_hbm, idx_v, tok_v, pos_v, sem):
    wid = lax.axis_index("s") * NC + lax.axis_index("c")
    base = wid * RPW
    pos_base = lax.rem(base, T)

    pltpu.sync_copy(idx_hbm.at[pl.ds(wid * NG, NG)], idx_v)
    cps = [
        pltpu.async_copy(
            tok_hbm.at[idx_v.at[g]], tok_v.at[pl.ds(g * GCHUNK, GCHUNK)], sem
        )
        for g in range(NG)
    ]
    pltpu.sync_copy(pos_hbm.at[pl.ds(pos_base, RPW)], pos_v)
    for cp in cps:
        cp.wait()

    def row_add(r, carry):
        for c in range(D // LANES):
            sl = pl.ds(c * LANES, LANES)
            plsc.addupdate(tok_v.at[r, sl], pos_v[r, sl])
        return carry

    lax.fori_loop(0, RPW, row_add, 0)

    pltpu.sync_copy(tok_v, out_hbm.at[pl.ds(base, RPW)])


@jax.jit
def kernel(x, tok_table, pos_table):
    idx = jnp.reshape(x.astype(jnp.int32), (ROWS // GCHUNK, GCHUNK))
    run = pl.kernel(
        _emb_body,
        out_type=jax.ShapeDtypeStruct((ROWS, D), jnp.float32),
        mesh=plsc.VectorSubcoreMesh(core_axis_name="c", subcore_axis_name="s"),
        scratch_types=[
            pltpu.VMEM((NG, GCHUNK), jnp.int32),
            pltpu.VMEM((RPW, D), jnp.float32),
            pltpu.VMEM((RPW, D), jnp.float32),
            pltpu.SemaphoreType.DMA,
        ],
    )
    out = run(idx, tok_table, pos_table)
    return jnp.reshape(out, (B, T, D))


# speedup vs baseline: 1.2756x; 1.0042x over previous
import functools

import jax
import jax.numpy as jnp
from jax import lax
from jax.experimental import pallas as pl
from jax.experimental.pallas import tpu as pltpu
from jax.experimental.pallas import tpu_sc as plsc

B = 4
T = 2048
D = 128
ROWS = B * T
NC = 2
NS = 16
NW = NC * NS
RPW = ROWS // NW
GCHUNK = 128
NG = RPW // GCHUNK
LANES = 16


def _emb_body(idx_hbm, tok_hbm, pos_hbm, out_hbm, idx_v, tok_v, pos_v, gsem, psem, osem):
    wid = lax.axis_index("s") * NC + lax.axis_index("c")
    base = wid * RPW
    pos_base = lax.rem(base, T)

    pltpu.sync_copy(idx_hbm.at[pl.ds(wid * NG, NG)], idx_v)
    gcps = [
        pltpu.async_copy(
            tok_hbm.at[idx_v.at[g]], tok_v.at[pl.ds(g * GCHUNK, GCHUNK)], gsem
        )
        for g in range(NG)
    ]
    pltpu.async_copy(pos_hbm.at[pl.ds(pos_base, RPW)], pos_v, psem).wait()

    ocps = []
    for g in range(NG):
        gcps[g].wait()

        @plsc.parallel_loop(g * GCHUNK, (g + 1) * GCHUNK, 1, unroll=4)
        def row_add(r):
            for c in range(D // LANES):
                sl = pl.ds(c * LANES, LANES)
                plsc.addupdate(tok_v.at[r, sl], pos_v[r, sl])

        ocps.append(
            pltpu.async_copy(
                tok_v.at[pl.ds(g * GCHUNK, GCHUNK)],
                out_hbm.at[pl.ds(base + g * GCHUNK, GCHUNK)],
                osem,
            )
        )
    for cp in ocps:
        cp.wait()


@jax.jit
def kernel(x, tok_table, pos_table):
    idx = jnp.reshape(x.astype(jnp.int32), (ROWS // GCHUNK, GCHUNK))
    run = pl.kernel(
        _emb_body,
        out_type=jax.ShapeDtypeStruct((ROWS, D), jnp.float32),
        mesh=plsc.VectorSubcoreMesh(core_axis_name="c", subcore_axis_name="s"),
        scratch_types=[
            pltpu.VMEM((NG, GCHUNK), jnp.int32),
            pltpu.VMEM((RPW, D), jnp.float32),
            pltpu.VMEM((RPW, D), jnp.float32),
            pltpu.SemaphoreType.DMA,
            pltpu.SemaphoreType.DMA,
            pltpu.SemaphoreType.DMA,
        ],
    )
    out = run(idx, tok_table, pos_table)
    return jnp.reshape(out, (B, T, D))


# speedup vs baseline: 1.2894x; 1.0108x over previous
import jax
import jax.numpy as jnp
from jax import lax
from jax.experimental import pallas as pl
from jax.experimental.pallas import tpu as pltpu
from jax.experimental.pallas import tpu_sc as plsc

B = 4
T = 2048
D = 128
ROWS = B * T
NC = 2
NS = 16
NW = NC * NS
RPW = ROWS // NW
WPB = T // RPW
GCHUNK = 128
NG = RPW // GCHUNK
LANES = 16


def _emb_body(x_hbm, tok_hbm, pos_hbm, out_hbm, idx_v, tok_v, pos_v, gsem, psem, osem):
    wid = lax.axis_index("s") * NC + lax.axis_index("c")
    base = wid * RPW
    b = lax.div(wid, WPB)
    off = lax.rem(wid, WPB) * RPW

    pcp = pltpu.async_copy(pos_hbm.at[pl.ds(off, RPW)], pos_v, psem)
    pltpu.sync_copy(x_hbm.at[b, pl.ds(off, RPW)], idx_v)
    gcps = [
        pltpu.async_copy(
            tok_hbm.at[idx_v.at[pl.ds(g * GCHUNK, GCHUNK)]],
            tok_v.at[pl.ds(g * GCHUNK, GCHUNK)],
            gsem,
        )
        for g in range(NG)
    ]
    pcp.wait()

    ocps = []
    for g in range(NG):
        gcps[g].wait()

        @plsc.parallel_loop(g * GCHUNK, (g + 1) * GCHUNK, 1, unroll=4)
        def row_add(r):
            for c in range(D // LANES):
                sl = pl.ds(c * LANES, LANES)
                plsc.addupdate(tok_v.at[r, sl], pos_v[r, sl])

        ocps.append(
            pltpu.async_copy(
                tok_v.at[pl.ds(g * GCHUNK, GCHUNK)],
                out_hbm.at[pl.ds(base + g * GCHUNK, GCHUNK)],
                osem,
            )
        )
    for cp in ocps:
        cp.wait()


@jax.jit
def kernel(x, tok_table, pos_table):
    run = pl.kernel(
        _emb_body,
        out_type=jax.ShapeDtypeStruct((ROWS, D), jnp.float32),
        mesh=plsc.VectorSubcoreMesh(core_axis_name="c", subcore_axis_name="s"),
        scratch_types=[
            pltpu.VMEM((RPW,), jnp.int32),
            pltpu.VMEM((RPW, D), jnp.float32),
            pltpu.VMEM((RPW, D), jnp.float32),
            pltpu.SemaphoreType.DMA,
            pltpu.SemaphoreType.DMA,
            pltpu.SemaphoreType.DMA,
        ],
    )
    out = run(x.astype(jnp.int32), tok_table, pos_table)
    return jnp.reshape(out, (B, T, D))


# speedup vs baseline: 1.3717x; 1.0639x over previous
import jax
import jax.numpy as jnp
from jax import lax
from jax.experimental import pallas as pl
from jax.experimental.pallas import tpu as pltpu
from jax.experimental.pallas import tpu_sc as plsc

B = 4
T = 2048
D = 128
NC = 2
NS = 16
NW = NC * NS
PPW = T // NW
LANES = 16


def _emb_body(x_hbm, tok_hbm, pos_hbm, out_hbm, idx_v, tok_v, pos_v, gsem, psem, osem, isem):
    wid = lax.axis_index("s") * NC + lax.axis_index("c")
    off = wid * PPW

    pcp = pltpu.async_copy(pos_hbm.at[pl.ds(off, PPW)], pos_v, psem)
    icps = [
        pltpu.async_copy(x_hbm.at[b, pl.ds(off, PPW)], idx_v.at[b], isem)
        for b in range(B)
    ]
    gcps = []
    for b in range(B):
        icps[b].wait()
        gcps.append(pltpu.async_copy(tok_hbm.at[idx_v.at[b]], tok_v.at[b], gsem))
    pcp.wait()

    ocps = []
    for b in range(B):
        gcps[b].wait()

        @plsc.parallel_loop(0, PPW, 1, unroll=4)
        def row_add(r):
            for c in range(D // LANES):
                sl = pl.ds(c * LANES, LANES)
                plsc.addupdate(tok_v.at[b, r, sl], pos_v[r, sl])

        ocps.append(
            pltpu.async_copy(tok_v.at[b], out_hbm.at[b, pl.ds(off, PPW), :], osem)
        )
    for cp in ocps:
        cp.wait()


@jax.jit
def kernel(x, tok_table, pos_table):
    run = pl.kernel(
        _emb_body,
        out_type=jax.ShapeDtypeStruct((B, T, D), jnp.float32),
        mesh=plsc.VectorSubcoreMesh(core_axis_name="c", subcore_axis_name="s"),
        scratch_types=[
            pltpu.VMEM((B, PPW), jnp.int32),
            pltpu.VMEM((B, PPW, D), jnp.float32),
            pltpu.VMEM((PPW, D), jnp.float32),
            pltpu.SemaphoreType.DMA,
            pltpu.SemaphoreType.DMA,
            pltpu.SemaphoreType.DMA,
            pltpu.SemaphoreType.DMA,
        ],
    )
    return run(x.astype(jnp.int32), tok_table, pos_table)


# speedup vs baseline: 1.4001x; 1.0207x over previous
import jax
import jax.numpy as jnp
from jax import lax
from jax.experimental import pallas as pl
from jax.experimental.pallas import tpu as pltpu
from jax.experimental.pallas import tpu_sc as plsc

B = 4
T = 2048
D = 128
NC = 2
NS = 16
NW = NC * NS
PPW = T // NW
LANES = 16


def _emb_body(x_hbm, tok_hbm, pos_hbm, out_hbm, idx_v, tok_v, pos_v, gsem, psem, osem, isem):
    wid = lax.axis_index("s") * NC + lax.axis_index("c")
    off = wid * PPW

    pcp = pltpu.async_copy(pos_hbm.at[pl.ds(off, PPW)], pos_v, psem)
    icps = [
        pltpu.async_copy(x_hbm.at[b, pl.ds(off, PPW)], idx_v.at[b], isem)
        for b in range(B)
    ]
    gcps = []
    for b in range(B):
        icps[b].wait()
        gcps.append(pltpu.async_copy(tok_hbm.at[idx_v.at[b]], tok_v.at[b], gsem))
    pcp.wait()

    ocps = []
    for b in range(B):
        gcps[b].wait()

        @plsc.parallel_loop(0, PPW, 1, unroll=2)
        def row_add(r):
            for c in range(D // LANES):
                sl = pl.ds(c * LANES, LANES)
                plsc.addupdate(tok_v.at[b, r, sl], pos_v[r, sl])

        ocps.append(
            pltpu.async_copy(tok_v.at[b], out_hbm.at[b, pl.ds(off, PPW), :], osem)
        )
    for cp in ocps:
        cp.wait()


@jax.jit
def kernel(x, tok_table, pos_table):
    run = pl.kernel(
        _emb_body,
        out_type=jax.ShapeDtypeStruct((B, T, D), jnp.float32),
        mesh=plsc.VectorSubcoreMesh(core_axis_name="c", subcore_axis_name="s"),
        scratch_types=[
            pltpu.VMEM((B, PPW), jnp.int32),
            pltpu.VMEM((B, PPW, D), jnp.float32),
            pltpu.VMEM((PPW, D), jnp.float32),
            pltpu.SemaphoreType.DMA,
            pltpu.SemaphoreType.DMA,
            pltpu.SemaphoreType.DMA,
            pltpu.SemaphoreType.DMA,
        ],
    )
    return run(x.astype(jnp.int32), tok_table, pos_table)


# speedup vs baseline: 1.4234x; 1.0166x over previous
import jax
import jax.numpy as jnp
from jax import lax
from jax.experimental import pallas as pl
from jax.experimental.pallas import tpu as pltpu
from jax.experimental.pallas import tpu_sc as plsc

B = 4
T = 2048
D = 128
NC = 2
NS = 16
NW = NC * NS
PPW = T // NW
LANES = 16


def _emb_body(x_hbm, tok_hbm, pos_hbm, out_hbm, idx_v, tok_v, pos_v, gsem, psem, osem, isem):
    wid = lax.axis_index("s") * NC + lax.axis_index("c")
    off = wid * PPW

    pcp = pltpu.async_copy(pos_hbm.at[pl.ds(off, PPW)], pos_v, psem)
    icps = [
        pltpu.async_copy(x_hbm.at[b, pl.ds(off, PPW)], idx_v.at[b], isem)
        for b in range(B)
    ]
    gcps = []
    for b in range(B):
        icps[b].wait()
        gcps.append(pltpu.async_copy(tok_hbm.at[idx_v.at[b]], tok_v.at[b], gsem))
    pcp.wait()

    ocps = []
    for b in range(B):
        gcps[b].wait()

        @plsc.parallel_loop(0, PPW, 1, unroll=1)
        def row_add(r):
            for c in range(D // LANES):
                sl = pl.ds(c * LANES, LANES)
                plsc.addupdate(tok_v.at[b, r, sl], pos_v[r, sl])

        ocps.append(
            pltpu.async_copy(tok_v.at[b], out_hbm.at[b, pl.ds(off, PPW), :], osem)
        )
    for cp in ocps:
        cp.wait()


@jax.jit
def kernel(x, tok_table, pos_table):
    run = pl.kernel(
        _emb_body,
        out_type=jax.ShapeDtypeStruct((B, T, D), jnp.float32),
        mesh=plsc.VectorSubcoreMesh(core_axis_name="c", subcore_axis_name="s"),
        scratch_types=[
            pltpu.VMEM((B, PPW), jnp.int32),
            pltpu.VMEM((B, PPW, D), jnp.float32),
            pltpu.VMEM((PPW, D), jnp.float32),
            pltpu.SemaphoreType.DMA,
            pltpu.SemaphoreType.DMA,
            pltpu.SemaphoreType.DMA,
            pltpu.SemaphoreType.DMA,
        ],
    )
    return run(x.astype(jnp.int32), tok_table, pos_table)
